# baseline (speedup 1.0000x reference)
import jax
import jax.numpy as jnp
from jax import lax
from jax.experimental import pallas as pl
from jax.experimental.pallas import tpu as pltpu

N_GENES = 100000
N_LATENT = 16
N_COMP = 32
B = 256
G_OI = 500

_ROW = N_LATENT * N_COMP
_G_STEP = 32
_G_PAD = 512
_RHO_CHUNK = 4096


def _gather_body(idx_ref, *refs):
    bs, out_ref = refs[:-1], refs[-1]
    i = pl.program_id(0)
    lane_iota = lax.broadcasted_iota(jnp.int32, (1, 128), 1)
    for j, blk in enumerate(bs):
        lane = idx_ref[jnp.minimum(_G_STEP * i + j, G_OI - 1)] & 127
        onehot = (lane_iota == lane).astype(jnp.float32)
        out_ref[j, :] = lax.dot_general(
            onehot, blk[...], (((1,), (1,)), ((), ())),
            preferred_element_type=jnp.float32)[0]


def _logit_body(lat_ref, w_ref, out_ref):
    out_ref[...] = lax.dot_general(
        w_ref[...], lat_ref[...], (((1,), (0,)), ((), ())),
        preferred_element_type=jnp.float32)


def _rho_body(lat_ref, w_ref, out_ref):
    out_ref[...] = lax.dot_general(
        w_ref[...], lat_ref[...], (((0,), (0,)), ((), ())),
        preferred_element_type=jnp.float32)


def kernel(latent, genes_oi, logit_weight_table, rho_weight_table):
    latT = latent.T
    wT = rho_weight_table.T
    t2 = jnp.transpose(logit_weight_table, (1, 2, 0)).reshape(_ROW, N_GENES)

    tbl_spec = [
        pl.BlockSpec(
            (_ROW, 128),
            (lambda j: lambda i, idx_ref: (0, idx_ref[jnp.minimum(_G_STEP * i + j, G_OI - 1)] >> 7))(j),
        )
        for j in range(_G_STEP)
    ]
    w_rows = pl.pallas_call(
        _gather_body,
        grid_spec=pltpu.PrefetchScalarGridSpec(
            num_scalar_prefetch=1,
            grid=(_G_PAD // _G_STEP,),
            in_specs=tbl_spec,
            out_specs=pl.BlockSpec((_G_STEP, _ROW), lambda i, idx_ref: (i, 0)),
        ),
        out_shape=jax.ShapeDtypeStruct((G_OI, _ROW), jnp.float32),
    )(genes_oi, *([t2] * _G_STEP))
    w3 = w_rows.reshape(G_OI, N_LATENT, N_COMP)

    logit_t = pl.pallas_call(
        _logit_body,
        grid=(4,),
        in_specs=[
            pl.BlockSpec((N_LATENT, B), lambda i: (0, 0)),
            pl.BlockSpec((128, N_LATENT, N_COMP), lambda i: (i, 0, 0)),
        ],
        out_specs=pl.BlockSpec((128, N_COMP, B), lambda i: (i, 0, 0)),
        out_shape=jax.ShapeDtypeStruct((G_OI, N_COMP, B), jnp.float32),
    )(latT, w3)

    n_chunks = pl.cdiv(N_GENES, _RHO_CHUNK)
    rho_t = pl.pallas_call(
        _rho_body,
        grid=(n_chunks,),
        in_specs=[
            pl.BlockSpec((N_LATENT, B), lambda i: (0, 0)),
            pl.BlockSpec((N_LATENT, _RHO_CHUNK), lambda i: (0, i)),
        ],
        out_specs=pl.BlockSpec((_RHO_CHUNK, B), lambda i: (i, 0)),
        out_shape=jax.ShapeDtypeStruct((N_GENES, B), jnp.float32),
    )(latT, wT)

    return (jnp.transpose(logit_t, (2, 0, 1)), rho_t.T)


# speedup vs baseline: 1.0481x; 1.0481x over previous
import jax
import jax.numpy as jnp
from jax import lax
from jax.experimental import pallas as pl
from jax.experimental.pallas import tpu as pltpu

N_GENES = 100000
N_LATENT = 16
N_COMP = 32
B = 256
G_OI = 500

_ROW = N_LATENT * N_COMP
_G_STEP = 64
_G_PAD = 512
_RHO_CHUNK = 8192


def _gather_body(idx_ref, *refs):
    bs, out_ref = refs[:-1], refs[-1]
    i = pl.program_id(0)
    lane_iota = lax.broadcasted_iota(jnp.int32, (1, 128), 1)
    for j, blk in enumerate(bs):
        lane = idx_ref[jnp.minimum(_G_STEP * i + j, G_OI - 1)] & 127
        onehot = (lane_iota == lane).astype(jnp.float32)
        out_ref[j, :] = lax.dot_general(
            onehot, blk[...], (((1,), (1,)), ((), ())),
            preferred_element_type=jnp.float32)[0]


def _logit_body(lat_ref, w_ref, out_ref):
    out_ref[...] = lax.dot_general(
        w_ref[...], lat_ref[...], (((1,), (0,)), ((), ())),
        preferred_element_type=jnp.float32)


def _rho_body(lat_ref, w_ref, out_ref):
    out_ref[...] = lax.dot_general(
        w_ref[...], lat_ref[...], (((0,), (0,)), ((), ())),
        preferred_element_type=jnp.float32)


def kernel(latent, genes_oi, logit_weight_table, rho_weight_table):
    latT = latent.T
    wT = rho_weight_table.T
    t2 = jnp.transpose(logit_weight_table, (1, 2, 0)).reshape(_ROW, N_GENES)

    tbl_spec = [
        pl.BlockSpec(
            (_ROW, 128),
            (lambda j: lambda i, idx_ref: (0, idx_ref[jnp.minimum(_G_STEP * i + j, G_OI - 1)] >> 7))(j),
        )
        for j in range(_G_STEP)
    ]
    w_rows = pl.pallas_call(
        _gather_body,
        grid_spec=pltpu.PrefetchScalarGridSpec(
            num_scalar_prefetch=1,
            grid=(_G_PAD // _G_STEP,),
            in_specs=tbl_spec,
            out_specs=pl.BlockSpec((_G_STEP, _ROW), lambda i, idx_ref: (i, 0)),
        ),
        out_shape=jax.ShapeDtypeStruct((G_OI, _ROW), jnp.float32),
    )(genes_oi, *([t2] * _G_STEP))
    w3 = w_rows.reshape(G_OI, N_LATENT, N_COMP)

    logit_t = pl.pallas_call(
        _logit_body,
        grid=(4,),
        in_specs=[
            pl.BlockSpec((N_LATENT, B), lambda i: (0, 0)),
            pl.BlockSpec((128, N_LATENT, N_COMP), lambda i: (i, 0, 0)),
        ],
        out_specs=pl.BlockSpec((128, N_COMP, B), lambda i: (i, 0, 0)),
        out_shape=jax.ShapeDtypeStruct((G_OI, N_COMP, B), jnp.float32),
    )(latT, w3)

    n_chunks = pl.cdiv(N_GENES, _RHO_CHUNK)
    rho_t = pl.pallas_call(
        _rho_body,
        grid=(n_chunks,),
        in_specs=[
            pl.BlockSpec((N_LATENT, B), lambda i: (0, 0)),
            pl.BlockSpec((N_LATENT, _RHO_CHUNK), lambda i: (0, i)),
        ],
        out_specs=pl.BlockSpec((_RHO_CHUNK, B), lambda i: (i, 0)),
        out_shape=jax.ShapeDtypeStruct((N_GENES, B), jnp.float32),
    )(latT, wT)

    return (jnp.transpose(logit_t, (2, 0, 1)), rho_t.T)
